# baseline (speedup 1.0000x reference)
import functools

import jax
import jax.numpy as jnp
import numpy as np
from jax import lax
from jax.experimental import pallas as pl
from jax.experimental.pallas import tpu as pltpu
from jax.experimental.pallas import tpu_sc as plsc

MAX_FRAMES = 1024
D = 256
NUM_FEATS = 10
HALF = D + 2 * NUM_FEATS
ROW = 2 * HALF
CHUNK = 64
NBUF = 2
LOOKAHEAD = 1
LUT_N = 2 * MAX_FRAMES + 512

_SIN_LUT = np.sin(np.pi * np.arange(LUT_N, dtype=np.float64) / MAX_FRAMES)
_SIN_LUT = _SIN_LUT.astype(np.float32)


def _make_kernel(B: int, n_workers: int):
    rows_per_w = B // n_workers
    n_chunks = rows_per_w // CHUNK
    assert rows_per_w % CHUNK == 0

    mesh = plsc.VectorSubcoreMesh(core_axis_name="c", subcore_axis_name="s")
    nc = plsc.get_sparse_core_info().num_cores

    scratch = [
        pltpu.VMEM((3 * rows_per_w,), jnp.int32),
        pltpu.VMEM((LUT_N,), jnp.float32),
    ]
    scratch += [pltpu.VMEM((CHUNK,), jnp.int32) for _ in range(NBUF)]
    scratch += [pltpu.VMEM((CHUNK,), jnp.int32) for _ in range(NBUF)]
    scratch += [pltpu.VMEM((CHUNK, ROW), jnp.float32) for _ in range(NBUF)]
    scratch += [pltpu.SemaphoreType.DMA for _ in range(3 * NBUF)]
    scratch += [pltpu.VMEM_SHARED((16, CHUNK // 2, ROW), jnp.float32)]
    scratch += [pltpu.SemaphoreType.DMA, pltpu.SemaphoreType.DMA]

    @functools.partial(
        pl.kernel,
        mesh=mesh,
        out_type=jax.ShapeDtypeStruct((B, ROW), jnp.float32),
        compiler_params=pltpu.CompilerParams(needs_layout_passes=False),
        scratch_types=scratch,
    )
    def enc(t_hbm, table_hbm, lut_hbm, out_hbm, tbuf, lutbuf, *bufs):
        ias = bufs[0:NBUF]
        ibs = bufs[NBUF:2 * NBUF]
        abufs = bufs[2 * NBUF:3 * NBUF]
        sas = bufs[3 * NBUF:4 * NBUF]
        sbs = bufs[4 * NBUF:5 * NBUF]
        sos = bufs[5 * NBUF:6 * NBUF]
        spm = bufs[6 * NBUF]
        sh1 = bufs[6 * NBUF + 1]
        sh2 = bufs[6 * NBUF + 2]
        sid = lax.axis_index("s")
        hop2_cp = [None]

        wid = lax.axis_index("s") * nc + lax.axis_index("c")
        base_row = wid * rows_per_w

        pltpu.sync_copy(lut_hbm, lutbuf)
        pltpu.sync_copy(t_hbm.at[pl.ds(base_row * 3, rows_per_w * 3)], tbuf)

        lane = lax.iota(jnp.int32, 16)
        gather_cps = [None] * NBUF
        out_cps = [None] * NBUF

        def launch(i):
            p = i % NBUF
            ia, ib, buf = ias[p], ibs[p], abufs[p]

            def diff_body(g, _, ia=ia, ib=ib, ch_off=i * CHUNK):
                r = g * 16
                f = 3 * (ch_off + r) + 3 * lane
                a = plsc.load_gather(tbuf, [f])
                b = plsc.load_gather(tbuf, [f + 1])
                c = plsc.load_gather(tbuf, [f + 2])
                ia[pl.ds(r, 16)] = b - a
                ib[pl.ds(r, 16)] = c - b
                return 0

            lax.fori_loop(0, CHUNK // 16, diff_body, 0)
            cpa = pltpu.async_copy(
                table_hbm.at[ia], buf.at[:, pl.ds(0, D)], sas[p])
            cpb = pltpu.async_copy(
                table_hbm.at[ib], buf.at[:, pl.ds(D, D)], sbs[p])
            gather_cps[p] = (cpa, cpb)

        def finish(i):
            p = i % NBUF
            ia, ib, buf = ias[p], ibs[p], abufs[p]
            cpa, cpb = gather_cps[p]

            cpb.wait()

            def reloc_body(r, _, buf=buf):
                rr = jnp.full((16,), r, dtype=jnp.int32)
                for i2 in range(D // 16 - 1, -1, -1):
                    v = buf[r, pl.ds(D + 16 * i2, 16)]
                    plsc.store_scatter(buf, [rr, HALF + 16 * i2 + lane], v)
                return 0

            lax.fori_loop(0, CHUNK, reloc_body, 0)

            def four_body(g, _, ia=ia, ib=ib, buf=buf):
                r = g * 16
                rows16 = r + lane
                d0 = ia[pl.ds(r, 16)]
                d1 = ib[pl.ds(r, 16)]
                for k in range(NUM_FEATS):
                    m0 = (d0 << k) & (2 * MAX_FRAMES - 1)
                    m1 = (d1 << k) & (2 * MAX_FRAMES - 1)
                    col = jnp.full((16,), D + k, dtype=jnp.int32)
                    plsc.store_scatter(
                        buf, [rows16, col], plsc.load_gather(lutbuf, [m0]))
                    plsc.store_scatter(
                        buf, [rows16, col + NUM_FEATS],
                        plsc.load_gather(lutbuf, [m0 + 512]))
                    plsc.store_scatter(
                        buf, [rows16, col + HALF],
                        plsc.load_gather(lutbuf, [m1]))
                    plsc.store_scatter(
                        buf, [rows16, col + HALF + NUM_FEATS],
                        plsc.load_gather(lutbuf, [m1 + 512]))
                return 0

            lax.fori_loop(0, CHUNK // 16, four_body, 0)

            cpa.wait()
            if i % 2 == 0:
                out_cps[p] = pltpu.async_copy(
                    buf, out_hbm.at[pl.ds(base_row + i * CHUNK, CHUNK)],
                    sos[p])
            else:
                for h in range(2):
                    if hop2_cp[0] is not None:
                        hop2_cp[0].wait()
                    pltpu.async_copy(
                        buf.at[pl.ds(h * (CHUNK // 2), CHUNK // 2)],
                        spm.at[sid], sh1).wait()
                    hop2_cp[0] = pltpu.async_copy(
                        spm.at[sid],
                        out_hbm.at[pl.ds(
                            base_row + i * CHUNK + h * (CHUNK // 2),
                            CHUNK // 2)], sh2)
                out_cps[p] = None

        for i in range(min(LOOKAHEAD, n_chunks)):
            launch(i)
        for i in range(n_chunks):
            nxt = i + LOOKAHEAD
            if nxt < n_chunks:
                if out_cps[nxt % NBUF] is not None:
                    out_cps[nxt % NBUF].wait()
                launch(nxt)
            finish(i)

        for cp in out_cps:
            if cp is not None:
                cp.wait()
        if hop2_cp[0] is not None:
            hop2_cp[0].wait()

    return enc


def kernel(t, embed_table):
    B = t.shape[0]
    t_flat = t.reshape(-1).astype(jnp.int32)
    lut = jnp.asarray(_SIN_LUT)
    enc = _make_kernel(B, 32)
    return enc(t_flat, embed_table, lut)


# speedup vs baseline: 1.2420x; 1.2420x over previous
import functools

import jax
import jax.numpy as jnp
import numpy as np
from jax import lax
from jax.experimental import pallas as pl
from jax.experimental.pallas import tpu as pltpu
from jax.experimental.pallas import tpu_sc as plsc

MAX_FRAMES = 1024
D = 256
NUM_FEATS = 10
HALF = D + 2 * NUM_FEATS
ROW = 2 * HALF
CHUNK = 64
NBUF = 2
LOOKAHEAD = 1
LUT_N = 2 * MAX_FRAMES + 512

_SIN_LUT = np.sin(np.pi * np.arange(LUT_N, dtype=np.float64) / MAX_FRAMES)
_SIN_LUT = _SIN_LUT.astype(np.float32)


def _make_kernel(B: int, n_workers: int):
    rows_per_w = B // n_workers
    n_chunks = rows_per_w // CHUNK
    assert rows_per_w % CHUNK == 0

    mesh = plsc.VectorSubcoreMesh(core_axis_name="c", subcore_axis_name="s")
    nc = plsc.get_sparse_core_info().num_cores

    scratch = [
        pltpu.VMEM((3 * rows_per_w,), jnp.int32),
        pltpu.VMEM((LUT_N,), jnp.float32),
    ]
    scratch += [pltpu.VMEM((CHUNK,), jnp.int32) for _ in range(NBUF)]
    scratch += [pltpu.VMEM((CHUNK,), jnp.int32) for _ in range(NBUF)]
    scratch += [pltpu.VMEM((CHUNK, ROW), jnp.float32) for _ in range(NBUF)]
    scratch += [pltpu.VMEM((CHUNK, D), jnp.float32) for _ in range(NBUF)]
    scratch += [pltpu.SemaphoreType.DMA for _ in range(3 * NBUF)]

    @functools.partial(
        pl.kernel,
        mesh=mesh,
        out_type=jax.ShapeDtypeStruct((B, ROW), jnp.float32),
        compiler_params=pltpu.CompilerParams(needs_layout_passes=False),
        scratch_types=scratch,
    )
    def enc(t_hbm, table_hbm, lut_hbm, out_hbm, tbuf, lutbuf, *bufs):
        ias = bufs[0:NBUF]
        ibs = bufs[NBUF:2 * NBUF]
        abufs = bufs[2 * NBUF:3 * NBUF]
        sts = bufs[3 * NBUF:4 * NBUF]
        sas = bufs[4 * NBUF:5 * NBUF]
        sbs = bufs[5 * NBUF:6 * NBUF]
        sos = bufs[6 * NBUF:7 * NBUF]

        wid = lax.axis_index("s") * nc + lax.axis_index("c")
        base_row = wid * rows_per_w

        pltpu.sync_copy(lut_hbm, lutbuf)
        pltpu.sync_copy(t_hbm.at[pl.ds(base_row * 3, rows_per_w * 3)], tbuf)

        lane = lax.iota(jnp.int32, 16)
        gather_cps = [None] * NBUF
        out_cps = [None] * NBUF

        def launch(i):
            p = i % NBUF
            ia, ib, buf, st = ias[p], ibs[p], abufs[p], sts[p]

            def diff_body(g, _, ia=ia, ib=ib, ch_off=i * CHUNK):
                r = g * 16
                f = 3 * (ch_off + r) + 3 * lane
                a = plsc.load_gather(tbuf, [f])
                b = plsc.load_gather(tbuf, [f + 1])
                c = plsc.load_gather(tbuf, [f + 2])
                ia[pl.ds(r, 16)] = b - a
                ib[pl.ds(r, 16)] = c - b
                return 0

            lax.fori_loop(0, CHUNK // 16, diff_body, 0)
            cpa = pltpu.async_copy(
                table_hbm.at[ia], buf.at[:, pl.ds(0, D)], sas[p])
            cpb = pltpu.async_copy(table_hbm.at[ib], st, sbs[p])
            gather_cps[p] = (cpa, cpb)

        def finish(i):
            p = i % NBUF
            ia, ib, buf, st = ias[p], ibs[p], abufs[p], sts[p]
            cpa, cpb = gather_cps[p]

            def four_body(g, _, ia=ia, ib=ib, buf=buf):
                r = g * 16
                rows16 = r + lane
                d0 = ia[pl.ds(r, 16)]
                d1 = ib[pl.ds(r, 16)]
                for k in range(NUM_FEATS):
                    m0 = (d0 << k) & (2 * MAX_FRAMES - 1)
                    m1 = (d1 << k) & (2 * MAX_FRAMES - 1)
                    col = jnp.full((16,), D + k, dtype=jnp.int32)
                    plsc.store_scatter(
                        buf, [rows16, col], plsc.load_gather(lutbuf, [m0]))
                    plsc.store_scatter(
                        buf, [rows16, col + NUM_FEATS],
                        plsc.load_gather(lutbuf, [m0 + 512]))
                    plsc.store_scatter(
                        buf, [rows16, col + HALF],
                        plsc.load_gather(lutbuf, [m1]))
                    plsc.store_scatter(
                        buf, [rows16, col + HALF + NUM_FEATS],
                        plsc.load_gather(lutbuf, [m1 + 512]))
                return 0

            lax.fori_loop(0, CHUNK // 16, four_body, 0)

            cpb.wait()

            @plsc.parallel_loop(0, CHUNK, unroll=2)
            def reloc_body(r, buf=buf, st=st):
                rr = jnp.full((16,), r, dtype=jnp.int32)
                for i2 in range(D // 16):
                    v = st[r, pl.ds(16 * i2, 16)]
                    plsc.store_scatter(buf, [rr, HALF + 16 * i2 + lane], v)

            cpa.wait()
            out_cps[p] = pltpu.async_copy(
                buf, out_hbm.at[pl.ds(base_row + i * CHUNK, CHUNK)], sos[p])

        for i in range(min(LOOKAHEAD, n_chunks)):
            launch(i)
        for i in range(n_chunks):
            nxt = i + LOOKAHEAD
            if nxt < n_chunks:
                if out_cps[nxt % NBUF] is not None:
                    out_cps[nxt % NBUF].wait()
                launch(nxt)
            finish(i)

        for cp in out_cps:
            if cp is not None:
                cp.wait()

    return enc


def kernel(t, embed_table):
    B = t.shape[0]
    t_flat = t.reshape(-1).astype(jnp.int32)
    lut = jnp.asarray(_SIN_LUT)
    enc = _make_kernel(B, 32)
    return enc(t_flat, embed_table, lut)


# speedup vs baseline: 1.2618x; 1.0159x over previous
import functools

import jax
import jax.numpy as jnp
import numpy as np
from jax import lax
from jax.experimental import pallas as pl
from jax.experimental.pallas import tpu as pltpu
from jax.experimental.pallas import tpu_sc as plsc

MAX_FRAMES = 1024
D = 256
NUM_FEATS = 10
HALF = D + 2 * NUM_FEATS
ROW = 2 * HALF
CHUNK = 64
NBUF = 2
LOOKAHEAD = 1
LUT_N = 2 * MAX_FRAMES + 512

_SIN_LUT = np.sin(np.pi * np.arange(LUT_N, dtype=np.float64) / MAX_FRAMES)
_SIN_LUT = _SIN_LUT.astype(np.float32)


def _make_kernel(B: int, n_workers: int):
    rows_per_w = B // n_workers
    n_chunks = rows_per_w // CHUNK
    assert rows_per_w % CHUNK == 0

    mesh = plsc.VectorSubcoreMesh(core_axis_name="c", subcore_axis_name="s")
    nc = plsc.get_sparse_core_info().num_cores

    scratch = [
        pltpu.VMEM((3 * rows_per_w,), jnp.int32),
        pltpu.VMEM((LUT_N,), jnp.float32),
    ]
    scratch += [pltpu.VMEM((CHUNK,), jnp.int32) for _ in range(NBUF)]
    scratch += [pltpu.VMEM((CHUNK,), jnp.int32) for _ in range(NBUF)]
    scratch += [pltpu.VMEM((CHUNK, ROW), jnp.float32) for _ in range(NBUF)]
    scratch += [pltpu.VMEM((CHUNK, D), jnp.float32) for _ in range(NBUF)]
    scratch += [pltpu.SemaphoreType.DMA for _ in range(3 * NBUF)]

    @functools.partial(
        pl.kernel,
        mesh=mesh,
        out_type=jax.ShapeDtypeStruct((B, ROW), jnp.float32),
        compiler_params=pltpu.CompilerParams(needs_layout_passes=False),
        scratch_types=scratch,
    )
    def enc(t_hbm, table_hbm, lut_hbm, out_hbm, tbuf, lutbuf, *bufs):
        ias = bufs[0:NBUF]
        ibs = bufs[NBUF:2 * NBUF]
        abufs = bufs[2 * NBUF:3 * NBUF]
        sts = bufs[3 * NBUF:4 * NBUF]
        sas = bufs[4 * NBUF:5 * NBUF]
        sbs = bufs[5 * NBUF:6 * NBUF]
        sos = bufs[6 * NBUF:7 * NBUF]

        wid = lax.axis_index("s") * nc + lax.axis_index("c")
        base_row = wid * rows_per_w

        pltpu.sync_copy(lut_hbm, lutbuf)
        pltpu.sync_copy(t_hbm.at[pl.ds(base_row * 3, rows_per_w * 3)], tbuf)

        lane = lax.iota(jnp.int32, 16)
        gather_cps = [None] * NBUF
        out_cps = [None] * NBUF

        def launch(i):
            p = i % NBUF
            ia, ib, buf, st = ias[p], ibs[p], abufs[p], sts[p]

            @plsc.parallel_loop(0, CHUNK // 16, unroll=2)
            def diff_body(g, ia=ia, ib=ib, ch_off=i * CHUNK):
                r = g * 16
                f = 3 * (ch_off + r) + 3 * lane
                a = plsc.load_gather(tbuf, [f])
                b = plsc.load_gather(tbuf, [f + 1])
                c = plsc.load_gather(tbuf, [f + 2])
                ia[pl.ds(r, 16)] = b - a
                ib[pl.ds(r, 16)] = c - b
            cpa = pltpu.async_copy(
                table_hbm.at[ia], buf.at[:, pl.ds(0, D)], sas[p])
            cpb = pltpu.async_copy(table_hbm.at[ib], st, sbs[p])
            gather_cps[p] = (cpa, cpb)

        def finish(i):
            p = i % NBUF
            ia, ib, buf, st = ias[p], ibs[p], abufs[p], sts[p]
            cpa, cpb = gather_cps[p]

            @plsc.parallel_loop(0, CHUNK // 16, unroll=2)
            def four_body(g, ia=ia, ib=ib, buf=buf):
                r = g * 16
                rows16 = r + lane
                d0 = ia[pl.ds(r, 16)]
                d1 = ib[pl.ds(r, 16)]
                for k in range(NUM_FEATS):
                    m0 = (d0 << k) & (2 * MAX_FRAMES - 1)
                    m1 = (d1 << k) & (2 * MAX_FRAMES - 1)
                    col = jnp.full((16,), D + k, dtype=jnp.int32)
                    plsc.store_scatter(
                        buf, [rows16, col], plsc.load_gather(lutbuf, [m0]))
                    plsc.store_scatter(
                        buf, [rows16, col + NUM_FEATS],
                        plsc.load_gather(lutbuf, [m0 + 512]))
                    plsc.store_scatter(
                        buf, [rows16, col + HALF],
                        plsc.load_gather(lutbuf, [m1]))
                    plsc.store_scatter(
                        buf, [rows16, col + HALF + NUM_FEATS],
                        plsc.load_gather(lutbuf, [m1 + 512]))


            cpb.wait()

            @plsc.parallel_loop(0, CHUNK, unroll=4)
            def reloc_body(r, buf=buf, st=st):
                rr = jnp.full((16,), r, dtype=jnp.int32)
                for i2 in range(D // 16):
                    v = st[r, pl.ds(16 * i2, 16)]
                    plsc.store_scatter(buf, [rr, HALF + 16 * i2 + lane], v)

            cpa.wait()
            out_cps[p] = pltpu.async_copy(
                buf, out_hbm.at[pl.ds(base_row + i * CHUNK, CHUNK)], sos[p])

        for i in range(min(LOOKAHEAD, n_chunks)):
            launch(i)
        for i in range(n_chunks):
            nxt = i + LOOKAHEAD
            if nxt < n_chunks:
                if out_cps[nxt % NBUF] is not None:
                    out_cps[nxt % NBUF].wait()
                launch(nxt)
            finish(i)

        for cp in out_cps:
            if cp is not None:
                cp.wait()

    return enc


def kernel(t, embed_table):
    B = t.shape[0]
    t_flat = t.reshape(-1).astype(jnp.int32)
    lut = jnp.asarray(_SIN_LUT)
    enc = _make_kernel(B, 32)
    return enc(t_flat, embed_table, lut)
